# Initial kernel scaffold; baseline (speedup 1.0000x reference)
import functools

import jax
import jax.numpy as jnp
from jax import lax
from jax.experimental import pallas as pl
from jax.experimental.pallas import tpu as pltpu
from jax.experimental.pallas import tpu_sc as plsc

_VOCAB = 50257
_D = 1024
_B = 4
_S = 2048
_NC = 2
_NS = 16
_NW = _NC * _NS
_S_PER_W = _S // _NW
_CHUNK = 32
_NCHUNK = _B * (_S_PER_W // _CHUNK)
_LANES = 16

_mesh = plsc.VectorSubcoreMesh(core_axis_name="c", subcore_axis_name="s")


def _body(tokens_hbm, table_hbm, pos_hbm, out_hbm,
          idx_v, pos_v, rows_v, g0, g1, w0, w1):
    cid = lax.axis_index("c")
    sid = lax.axis_index("s")
    wid = sid * _NC + cid
    s0 = wid * _S_PER_W

    for b in range(_B):
        pltpu.sync_copy(tokens_hbm.at[pl.ds(b * _S + s0, _S_PER_W)],
                        idx_v.at[b])

    gsems = (g0, g1)
    wsems = (w0, w1)

    def mk_gather(k):
        sc, b = k // _B, k % _B
        buf = k % 2
        return pltpu.make_async_copy(
            table_hbm.at[idx_v.at[b, pl.ds(sc * _CHUNK, _CHUNK)]],
            rows_v.at[buf],
            gsems[buf])

    def mk_write(k):
        sc, b = k // _B, k % _B
        buf = k % 2
        return pltpu.make_async_copy(
            rows_v.at[buf],
            out_hbm.at[pl.ds(b * _S + s0 + sc * _CHUNK, _CHUNK)],
            wsems[buf])

    def add_pos(rows):
        def row_body(r, carry):
            for c in range(_D // _LANES):
                v = pos_v[r, pl.ds(c * _LANES, _LANES)]
                plsc.addupdate(rows.at[r, pl.ds(c * _LANES, _LANES)], v)
            return carry
        lax.fori_loop(0, _CHUNK, row_body, 0)

    mk_gather(0).start()
    for k in range(_NCHUNK):
        sc, b = k // _B, k % _B
        buf = k % 2
        if b == 0:
            pltpu.sync_copy(pos_hbm.at[pl.ds(s0 + sc * _CHUNK, _CHUNK)],
                            pos_v)
        if k + 1 < _NCHUNK:
            if k >= 1:
                mk_write(k - 1).wait()
            mk_gather(k + 1).start()
        mk_gather(k).wait()
        add_pos(rows_v.at[buf])
        mk_write(k).start()
    mk_write(_NCHUNK - 2).wait()
    mk_write(_NCHUNK - 1).wait()


_emb = functools.partial(
    pl.kernel,
    out_type=jax.ShapeDtypeStruct((_B * _S, _D), jnp.float32),
    mesh=_mesh,
    scratch_types=[
        pltpu.VMEM((_B, _S_PER_W), jnp.int32),
        pltpu.VMEM((_CHUNK, _D), jnp.float32),
        pltpu.VMEM((2, _CHUNK, _D), jnp.float32),
        pltpu.SemaphoreType.DMA,
        pltpu.SemaphoreType.DMA,
        pltpu.SemaphoreType.DMA,
        pltpu.SemaphoreType.DMA,
    ],
)(_body)


@jax.jit
def kernel(tokens, token_table, pos_table):
    tokens_f = tokens.reshape(-1).astype(jnp.int32)
    out = _emb(tokens_f, token_table, pos_table)
    return out.reshape(_B, _S, _D)


# speedup vs baseline: 1.3306x; 1.3306x over previous
import functools

import jax
import jax.numpy as jnp
from jax import lax
from jax.experimental import pallas as pl
from jax.experimental.pallas import tpu as pltpu
from jax.experimental.pallas import tpu_sc as plsc

_VOCAB = 50257
_D = 1024
_B = 4
_S = 2048
_NC = 2
_NS = 16
_NW = _NC * _NS
_S_PER_W = _S // _NW
_CHUNK = 32
_NCHUNK = _B * (_S_PER_W // _CHUNK)
_LANES = 16


def _body(tokens_hbm, table_hbm, pos_hbm, out_hbm,
          idx_v, pos_v, rows_v, g0, g1, w0, w1):
    cid = lax.axis_index("c")
    sid = lax.axis_index("s")
    wid = sid * _NC + cid
    s0 = wid * _S_PER_W

    for b in range(_B):
        pltpu.sync_copy(tokens_hbm.at[pl.ds(b * _S + s0, _S_PER_W)],
                        idx_v.at[b])

    gsems = (g0, g1)
    wsems = (w0, w1)

    def mk_gather(k):
        sc, b = k // _B, k % _B
        buf = k % 2
        return pltpu.make_async_copy(
            table_hbm.at[idx_v.at[b, pl.ds(sc * _CHUNK, _CHUNK)]],
            rows_v.at[buf],
            gsems[buf])

    def mk_write(k):
        sc, b = k // _B, k % _B
        buf = k % 2
        return pltpu.make_async_copy(
            rows_v.at[buf],
            out_hbm.at[pl.ds(b * _S + s0 + sc * _CHUNK, _CHUNK)],
            wsems[buf])

    def add_pos(rows):
        def row_body(r, carry):
            for c in range(_D // _LANES):
                v = pos_v[r, pl.ds(c * _LANES, _LANES)]
                plsc.addupdate(rows.at[r, pl.ds(c * _LANES, _LANES)], v)
            return carry
        lax.fori_loop(0, _CHUNK, row_body, 0)

    mk_gather(0).start()
    for k in range(_NCHUNK):
        sc, b = k // _B, k % _B
        buf = k % 2
        if b == 0:
            pltpu.sync_copy(pos_hbm.at[pl.ds(s0 + sc * _CHUNK, _CHUNK)],
                            pos_v)
        if k + 1 < _NCHUNK:
            if k >= 1:
                mk_write(k - 1).wait()
            mk_gather(k + 1).start()
        mk_gather(k).wait()
        add_pos(rows_v.at[buf])
        mk_write(k).start()
    mk_write(_NCHUNK - 2).wait()
    mk_write(_NCHUNK - 1).wait()


_emb_cache = []


def _get_emb():
    if not _emb_cache:
        mesh = plsc.VectorSubcoreMesh(core_axis_name="c", subcore_axis_name="s",
                                      num_cores=_NC, num_subcores=_NS)
        emb = functools.partial(
            pl.kernel,
            out_type=jax.ShapeDtypeStruct((_B * _S, _D), jnp.float32),
            mesh=mesh,
            scratch_types=[
                pltpu.VMEM((_B, _S_PER_W), jnp.int32),
                pltpu.VMEM((_CHUNK, _D), jnp.float32),
                pltpu.VMEM((2, _CHUNK, _D), jnp.float32),
                pltpu.SemaphoreType.DMA,
                pltpu.SemaphoreType.DMA,
                pltpu.SemaphoreType.DMA,
                pltpu.SemaphoreType.DMA,
            ],
        )(_body)
        _emb_cache.append(emb)
    return _emb_cache[0]


@jax.jit
def kernel(tokens, token_table, pos_table):
    tokens_f = tokens.reshape(-1).astype(jnp.int32)
    out = _get_emb()(tokens_f, token_table, pos_table)
    return out.reshape(_B, _S, _D)
